# baseline (speedup 1.0000x reference)
import functools

import jax
import jax.numpy as jnp
from jax import lax
from jax.experimental import pallas as pl
from jax.experimental.pallas import tpu as pltpu
from jax.experimental.pallas import tpu_sc as plsc

B = 16384
SEQ = 51
T = 50
E = 32
NC = 2
NS = 16
NW = NC * NS

BPW = B // NW
RPW = BPW * T
CI = 1024
N_CHUNKS = RPW // CI
L = 16
IDX_W = 128


def _sc_gather(table, x):
    mesh = plsc.VectorSubcoreMesh(
        core_axis_name="c", subcore_axis_name="s", num_cores=NC, num_subcores=NS
    )

    @functools.partial(
        pl.kernel,
        mesh=mesh,
        compiler_params=pltpu.CompilerParams(use_tc_tiling_on_sc=False,
                                             needs_layout_passes=False),
        out_type=(
            jax.ShapeDtypeStruct((B * T, E), jnp.float32),
            jax.ShapeDtypeStruct((B, E), jnp.float32),
        ),
        scratch_types=[
            pltpu.VMEM((BPW, SEQ), jnp.int32),
            pltpu.VMEM((CI,), jnp.int32),
            pltpu.VMEM((CI,), jnp.int32),
            pltpu.VMEM((CI, E), jnp.float32),
            pltpu.VMEM((CI, E), jnp.float32),
            pltpu.VMEM((BPW,), jnp.int32),
            pltpu.VMEM((BPW, E), jnp.float32),
            pltpu.SemaphoreType.DMA,
            pltpu.SemaphoreType.DMA,
            pltpu.SemaphoreType.DMA,
            pltpu.SemaphoreType.DMA,
            pltpu.SemaphoreType.DMA,
            pltpu.SemaphoreType.DMA,
        ],
    )
    def k(table_hbm, x_hbm, ub_hbm, qa_hbm,
          x_v, ibuf0, ibuf1, rows0, rows1, ad_idx_v, ad_rows_v,
          gsem0, gsem1, osem0, osem1, adsem, aosem):
        wid = lax.axis_index("s") * NC + lax.axis_index("c")
        b0 = wid * BPW
        ibuf = (ibuf0, ibuf1)
        rows = (rows0, rows1)
        gsem = (gsem0, gsem1)
        osem = (osem0, osem1)
        lane = lax.iota(jnp.int32, L)

        pltpu.sync_copy(x_hbm.at[pl.ds(b0, BPW)], x_v)

        col_ad = jnp.full((L,), T, dtype=jnp.int32)
        for q in range(BPW // L):
            ad_idx_v[pl.ds(q * L, L)] = plsc.load_gather(
                x_v, [jnp.int32(q * L) + lane, col_ad])
        ad_descs = [
            pltpu.async_copy(table_hbm.at[ad_idx_v.at[pl.ds(j * IDX_W, IDX_W)]],
                             ad_rows_v.at[pl.ds(j * IDX_W, IDX_W)], adsem)
            for j in range(BPW // IDX_W)
        ]

        def build(g):
            buf = ibuf[g % 2]

            def body(q, carry):
                p = q * L + lane
                vals = plsc.load_gather(x_v, [p // T, p % T])
                plsc.store_scatter(buf, [p - jnp.int32(g * CI)], vals)
                return carry

            lax.fori_loop(g * CI // L, (g + 1) * CI // L, body, 0)

        def fire(g):
            p = g % 2
            return [
                pltpu.async_copy(table_hbm.at[ibuf[p].at[pl.ds(j * IDX_W,
                                                               IDX_W)]],
                                 rows[p].at[pl.ds(j * IDX_W, IDX_W)], gsem[p])
                for j in range(CI // IDX_W)
            ]

        out_descs = [None, None]
        build(0)
        gather_descs = fire(0)
        for g in range(N_CHUNKS):
            p = g % 2
            next_descs = None
            if g + 1 < N_CHUNKS:
                build(g + 1)
                if out_descs[(g + 1) % 2] is not None:
                    out_descs[(g + 1) % 2].wait()
                    out_descs[(g + 1) % 2] = None
                next_descs = fire(g + 1)
            for d in gather_descs:
                d.wait()
            gather_descs = next_descs
            out_descs[p] = pltpu.async_copy(
                rows[p], ub_hbm.at[pl.ds(wid * RPW + g * CI, CI)], osem[p])
        for d in out_descs:
            if d is not None:
                d.wait()

        for d in ad_descs:
            d.wait()
        pltpu.async_copy(ad_rows_v, qa_hbm.at[pl.ds(b0, BPW)], aosem).wait()

    return k(table, x)


def _mask_body(x_ref, o_ref):
    o_ref[:, :, 0] = (x_ref[:, :T] > 0).astype(jnp.float32)


def _tc_mask(x):
    blk = 256
    return pl.pallas_call(
        _mask_body,
        grid=(B // blk,),
        in_specs=[pl.BlockSpec((blk, SEQ), lambda i: (i, 0))],
        out_specs=pl.BlockSpec((blk, T, 1), lambda i: (i, 0, 0)),
        out_shape=jax.ShapeDtypeStruct((B, T, 1), jnp.float32),
    )(x)


def kernel(x, table):
    ub, qa = _sc_gather(table, x)
    mask = _tc_mask(x)
    return (qa.reshape(B, 1, E), ub.reshape(B, T, E), mask)


# speedup vs baseline: 2.5911x; 2.5911x over previous
import functools

import jax
import jax.numpy as jnp
from jax import lax
from jax.experimental import pallas as pl
from jax.experimental.pallas import tpu as pltpu
from jax.experimental.pallas import tpu_sc as plsc

B = 16384
SEQ = 51
T = 50
E = 32
NC = 2
NS = 16
NW = NC * NS

BPW = B // NW
Q = 128 // E
M = B // Q
IDX_W = 128
ND = BPW // IDX_W


def _sc_gather(table, xT):
    mesh = plsc.VectorSubcoreMesh(
        core_axis_name="c", subcore_axis_name="s", num_cores=NC, num_subcores=NS
    )

    @functools.partial(
        pl.kernel,
        mesh=mesh,
        compiler_params=pltpu.CompilerParams(use_tc_tiling_on_sc=False,
                                             needs_layout_passes=False),
        out_type=(
            jax.ShapeDtypeStruct((T * M, 128), jnp.float32),
            jax.ShapeDtypeStruct((M, 128), jnp.float32),
        ),
        scratch_types=[
            pltpu.VMEM((SEQ, BPW), jnp.int32),
            pltpu.VMEM((BPW, E), jnp.float32),
            pltpu.VMEM((BPW, E), jnp.float32),
            pltpu.SemaphoreType.DMA,
            pltpu.SemaphoreType.DMA,
            pltpu.SemaphoreType.DMA,
            pltpu.SemaphoreType.DMA,
        ],
    )
    def k(table_hbm, xT_hbm, iub_hbm, iqa_hbm,
          x_v, rows0, rows1,
          gsem0, gsem1, osem0, osem1):
        wid = lax.axis_index("s") * NC + lax.axis_index("c")
        b0 = wid * BPW
        q = wid // (NW // Q)
        m0 = (wid % (NW // Q)) * BPW
        rows = (rows0, rows1)
        gsem = (gsem0, gsem1)
        osem = (osem0, osem1)

        pltpu.sync_copy(xT_hbm.at[:, pl.ds(b0, BPW)], x_v)

        def fire(t):
            p = t % 2
            return [
                pltpu.async_copy(
                    table_hbm.at[x_v.at[t, pl.ds(j * IDX_W, IDX_W)]],
                    rows[p].at[pl.ds(j * IDX_W, IDX_W)], gsem[p])
                for j in range(ND)
            ]

        out_descs = [None, None]
        gather_descs = fire(0)
        for t in range(SEQ):
            p = t % 2
            next_descs = None
            if t + 1 < SEQ:
                if out_descs[(t + 1) % 2] is not None:
                    out_descs[(t + 1) % 2].wait()
                    out_descs[(t + 1) % 2] = None
                next_descs = fire(t + 1)
            for d in gather_descs:
                d.wait()
            gather_descs = next_descs
            if t < T:
                dst = iub_hbm.at[pl.ds(t * M + m0, BPW), pl.ds(q * E, E)]
            else:
                dst = iqa_hbm.at[pl.ds(m0, BPW), pl.ds(q * E, E)]
            out_descs[p] = pltpu.async_copy(rows[p], dst, osem[p])
        for d in out_descs:
            if d is not None:
                d.wait()

    return k(table, xT)


def _xpose_body(i_ref, o_ref):
    v = i_ref[...]
    for q in range(Q):
        o_ref[0, :, q * M:(q + 1) * M] = v[:, q * E:(q + 1) * E].T


def _tc_unpack(i128, nslots):
    return pl.pallas_call(
        _xpose_body,
        grid=(nslots,),
        in_specs=[pl.BlockSpec((M, 128), lambda t: (t, 0))],
        out_specs=pl.BlockSpec((1, E, B), lambda t: (t, 0, 0)),
        out_shape=jax.ShapeDtypeStruct((nslots, E, B), jnp.float32),
    )(i128)


def _mask_body(x_ref, o_ref):
    o_ref[...] = (x_ref[:T] > 0).astype(jnp.float32)


def _tc_mask(xT):
    return pl.pallas_call(
        _mask_body,
        grid=(1,),
        in_specs=[pl.BlockSpec((SEQ, B), lambda i: (0, 0))],
        out_specs=pl.BlockSpec((T, B), lambda i: (0, 0)),
        out_shape=jax.ShapeDtypeStruct((T, B), jnp.float32),
    )(xT)


def kernel(x, table):
    xT = jnp.swapaxes(x, 0, 1)
    iub, iqa = _sc_gather(table, xT)
    ub = _tc_unpack(iub, T)
    qa = _tc_unpack(iqa, 1)
    mask2d = _tc_mask(xT)
    return (qa.transpose(2, 0, 1),
            ub.transpose(2, 0, 1),
            mask2d.T[:, :, None])


# speedup vs baseline: 3.0865x; 1.1912x over previous
import functools

import jax
import jax.numpy as jnp
from jax import lax
from jax.experimental import pallas as pl
from jax.experimental.pallas import tpu as pltpu
from jax.experimental.pallas import tpu_sc as plsc

B = 16384
SEQ = 51
T = 50
E = 32
NC = 2
NS = 16
NW = NC * NS

BPW = B // NW
Q = 128 // E
M = B // Q
IDX_W = 128
ND = BPW // IDX_W


def _sc_gather(table, xT):
    mesh = plsc.VectorSubcoreMesh(
        core_axis_name="c", subcore_axis_name="s", num_cores=NC, num_subcores=NS
    )

    @functools.partial(
        pl.kernel,
        mesh=mesh,
        compiler_params=pltpu.CompilerParams(use_tc_tiling_on_sc=False,
                                             needs_layout_passes=False),
        out_type=(
            jax.ShapeDtypeStruct((T * M, 128), jnp.float32),
            jax.ShapeDtypeStruct((M, 128), jnp.float32),
        ),
        scratch_types=[
            pltpu.VMEM((SEQ, BPW), jnp.int32),
            pltpu.VMEM((BPW, E), jnp.float32),
            pltpu.VMEM((BPW, E), jnp.float32),
            pltpu.SemaphoreType.DMA,
            pltpu.SemaphoreType.DMA,
            pltpu.SemaphoreType.DMA,
            pltpu.SemaphoreType.DMA,
        ],
    )
    def k(table_hbm, xT_hbm, iub_hbm, iqa_hbm,
          x_v, rows0, rows1,
          gsem0, gsem1, osem0, osem1):
        wid = lax.axis_index("s") * NC + lax.axis_index("c")
        b0 = wid * BPW
        q = wid // (NW // Q)
        m0 = (wid % (NW // Q)) * BPW
        rows = (rows0, rows1)
        gsem = (gsem0, gsem1)
        osem = (osem0, osem1)

        pltpu.sync_copy(xT_hbm.at[:, pl.ds(b0, BPW)], x_v)

        def fire(t):
            p = t % 2
            return [
                pltpu.async_copy(
                    table_hbm.at[x_v.at[t, pl.ds(j * IDX_W, IDX_W)]],
                    rows[p].at[pl.ds(j * IDX_W, IDX_W)], gsem[p])
                for j in range(ND)
            ]

        out_descs = [None, None]
        gather_descs = fire(0)
        for t in range(SEQ):
            p = t % 2
            next_descs = None
            if t + 1 < SEQ:
                if out_descs[(t + 1) % 2] is not None:
                    out_descs[(t + 1) % 2].wait()
                    out_descs[(t + 1) % 2] = None
                next_descs = fire(t + 1)
            for d in gather_descs:
                d.wait()
            gather_descs = next_descs
            if t < T:
                dst = iub_hbm.at[pl.ds(t * M + m0, BPW), pl.ds(q * E, E)]
            else:
                dst = iqa_hbm.at[pl.ds(m0, BPW), pl.ds(q * E, E)]
            out_descs[p] = pltpu.async_copy(rows[p], dst, osem[p])
        for d in out_descs:
            if d is not None:
                d.wait()

    return k(table, xT)


def _xpose_body(iub_ref, iqa_ref, ub_ref, qa_ref):
    t = pl.program_id(0)

    def emit(i_ref, o_ref):
        vT = i_ref[...].T
        for q in range(Q):
            o_ref[0, :, q * M:(q + 1) * M] = vT[q * E:(q + 1) * E, :]

    @pl.when(t < T)
    def _():
        emit(iub_ref, ub_ref)

    @pl.when(t == T)
    def _():
        emit(iqa_ref, qa_ref)


def _tc_unpack(iub, iqa):
    return pl.pallas_call(
        _xpose_body,
        grid=(T + 1,),
        in_specs=[
            pl.BlockSpec((M, 128), lambda t: (jnp.minimum(t, T - 1), 0)),
            pl.BlockSpec((M, 128), lambda t: (0, 0)),
        ],
        out_specs=[
            pl.BlockSpec((1, E, B), lambda t: (jnp.minimum(t, T - 1), 0, 0)),
            pl.BlockSpec((1, E, B), lambda t: (0, 0, 0)),
        ],
        out_shape=(jax.ShapeDtypeStruct((T, E, B), jnp.float32),
                   jax.ShapeDtypeStruct((1, E, B), jnp.float32)),
    )(iub, iqa)


def _mask_body(x_ref, o_ref):
    o_ref[...] = (x_ref[:T] > 0).astype(jnp.float32)


def _tc_mask(xT):
    return pl.pallas_call(
        _mask_body,
        grid=(1,),
        in_specs=[pl.BlockSpec((SEQ, B), lambda i: (0, 0))],
        out_specs=pl.BlockSpec((T, B), lambda i: (0, 0)),
        out_shape=jax.ShapeDtypeStruct((T, B), jnp.float32),
    )(xT)


def kernel(x, table):
    xT = jnp.swapaxes(x, 0, 1)
    iub, iqa = _sc_gather(table, xT)
    ub, qa = _tc_unpack(iub, iqa)
    mask2d = _tc_mask(xT)
    return (qa.transpose(2, 0, 1),
            ub.transpose(2, 0, 1),
            mask2d.T[:, :, None])


# speedup vs baseline: 3.1436x; 1.0185x over previous
import functools

import jax
import jax.numpy as jnp
from jax import lax
from jax.experimental import pallas as pl
from jax.experimental.pallas import tpu as pltpu
from jax.experimental.pallas import tpu_sc as plsc

B = 16384
SEQ = 51
T = 50
E = 32
NC = 2
NS = 16
NW = NC * NS

BPW = B // NW
Q = 128 // E
M = B // Q
IDX_W = 128
ND = BPW // IDX_W


def _sc_gather(table, xT):
    mesh = plsc.VectorSubcoreMesh(
        core_axis_name="c", subcore_axis_name="s", num_cores=NC, num_subcores=NS
    )

    @functools.partial(
        pl.kernel,
        mesh=mesh,
        compiler_params=pltpu.CompilerParams(use_tc_tiling_on_sc=False,
                                             needs_layout_passes=False),
        out_type=(
            jax.ShapeDtypeStruct((T * M, 128), jnp.float32),
            jax.ShapeDtypeStruct((M, 128), jnp.float32),
        ),
        scratch_types=[
            pltpu.VMEM((SEQ, BPW), jnp.int32),
            pltpu.VMEM((BPW, E), jnp.float32),
            pltpu.VMEM((BPW, E), jnp.float32),
            pltpu.SemaphoreType.DMA,
            pltpu.SemaphoreType.DMA,
            pltpu.SemaphoreType.DMA,
            pltpu.SemaphoreType.DMA,
        ],
    )
    def k(table_hbm, xT_hbm, iub_hbm, iqa_hbm,
          x_v, rows0, rows1,
          gsem0, gsem1, osem0, osem1):
        wid = lax.axis_index("s") * NC + lax.axis_index("c")
        b0 = wid * BPW
        q = wid // (NW // Q)
        m0 = (wid % (NW // Q)) * BPW
        rows = (rows0, rows1)
        gsem = (gsem0, gsem1)
        osem = (osem0, osem1)

        pltpu.sync_copy(xT_hbm.at[:, pl.ds(b0, BPW)], x_v)

        def fire(t):
            p = t % 2
            return [
                pltpu.async_copy(
                    table_hbm.at[x_v.at[t, pl.ds(j * IDX_W, IDX_W)]],
                    rows[p].at[pl.ds(j * IDX_W, IDX_W)], gsem[p])
                for j in range(ND)
            ]

        out_descs = [None, None]
        gather_descs = fire(0)
        for t in range(SEQ):
            p = t % 2
            next_descs = None
            if t + 1 < SEQ:
                if out_descs[(t + 1) % 2] is not None:
                    out_descs[(t + 1) % 2].wait()
                    out_descs[(t + 1) % 2] = None
                next_descs = fire(t + 1)
            for d in gather_descs:
                d.wait()
            gather_descs = next_descs
            if t < T:
                dst = iub_hbm.at[pl.ds(t * M + m0, BPW), pl.ds(q * E, E)]
            else:
                dst = iqa_hbm.at[pl.ds(m0, BPW), pl.ds(q * E, E)]
            out_descs[p] = pltpu.async_copy(rows[p], dst, osem[p])
        for d in out_descs:
            if d is not None:
                d.wait()

    return k(table, xT)


SPS = 2
NSTEP = T // SPS


def _xpose_body(iub_ref, iqa_ref, ub_ref, qa_ref):
    t = pl.program_id(0)

    @pl.when(t < NSTEP)
    def _():
        vT = iub_ref[...].T
        for s in range(SPS):
            for q in range(Q):
                ub_ref[s, :, q * M:(q + 1) * M] = (
                    vT[q * E:(q + 1) * E, s * M:(s + 1) * M])

    @pl.when(t == NSTEP)
    def _():
        vT = iqa_ref[...].T
        for q in range(Q):
            qa_ref[0, :, q * M:(q + 1) * M] = vT[q * E:(q + 1) * E, :]


def _tc_unpack(iub, iqa):
    return pl.pallas_call(
        _xpose_body,
        grid=(NSTEP + 1,),
        in_specs=[
            pl.BlockSpec((SPS * M, 128), lambda t: (jnp.minimum(t, NSTEP - 1), 0)),
            pl.BlockSpec((M, 128), lambda t: (0, 0)),
        ],
        out_specs=[
            pl.BlockSpec((SPS, E, B), lambda t: (jnp.minimum(t, NSTEP - 1), 0, 0)),
            pl.BlockSpec((1, E, B), lambda t: (0, 0, 0)),
        ],
        out_shape=(jax.ShapeDtypeStruct((T, E, B), jnp.float32),
                   jax.ShapeDtypeStruct((1, E, B), jnp.float32)),
    )(iub, iqa)


def _mask_body(x_ref, o_ref):
    o_ref[...] = (x_ref[:T] > 0).astype(jnp.float32)


def _tc_mask(xT):
    return pl.pallas_call(
        _mask_body,
        grid=(1,),
        in_specs=[pl.BlockSpec((SEQ, B), lambda i: (0, 0))],
        out_specs=pl.BlockSpec((T, B), lambda i: (0, 0)),
        out_shape=jax.ShapeDtypeStruct((T, B), jnp.float32),
    )(xT)


def kernel(x, table):
    xT = jnp.swapaxes(x, 0, 1)
    iub, iqa = _sc_gather(table, xT)
    ub, qa = _tc_unpack(iub, iqa)
    mask2d = _tc_mask(xT)
    return (qa.transpose(2, 0, 1),
            ub.transpose(2, 0, 1),
            mask2d.T[:, :, None])
